# baseline (speedup 1.0000x reference)
import functools

import jax
import jax.numpy as jnp
from jax import lax
from jax.experimental import pallas as pl
from jax.experimental.pallas import tpu as pltpu
from jax.experimental.pallas import tpu_sc as plsc

_NC = 2
_NS = 16
_NW = _NC * _NS


def _sc_gather(X3, emb_E, emb_R):
    M = X3.shape[1] * X3.shape[2]
    D = emb_E.shape[1]
    bpw = M // _NW
    nchunk = bpw // 128

    mesh = plsc.VectorSubcoreMesh(core_axis_name="c", subcore_axis_name="s")

    @functools.partial(
        pl.kernel,
        mesh=mesh,
        out_type=[jax.ShapeDtypeStruct((M, D), jnp.float32) for _ in range(3)],
        scratch_types=[
            pltpu.VMEM((3, nchunk, 128), jnp.int32),
            pltpu.VMEM((3 * bpw, D), jnp.float32),
            pltpu.SemaphoreType.DMA,
            pltpu.SemaphoreType.DMA,
        ],
    )
    def gather_kernel(x_hbm, e_hbm, r_hbm, out_hs, out_ls, out_ts,
                      idx_v, rows_v, gsem, osem):
        wid = lax.axis_index("s") * _NC + lax.axis_index("c")
        rowbase = wid * nchunk
        base = wid * bpw
        pltpu.sync_copy(x_hbm.at[:, pl.ds(rowbase, nchunk), :], idx_v)
        tabs = (e_hbm, r_hbm, e_hbm)
        cps = []
        for seg in range(3):
            for j in range(nchunk):
                cps.append(pltpu.async_copy(
                    tabs[seg].at[idx_v.at[seg, j]],
                    rows_v.at[pl.ds((seg * nchunk + j) * 128, 128)], gsem))
        for cp in cps:
            cp.wait()
        outs = (out_hs, out_ls, out_ts)
        ocps = [
            pltpu.async_copy(rows_v.at[pl.ds(seg * bpw, bpw)],
                             outs[seg].at[pl.ds(base, bpw)], osem)
            for seg in range(3)
        ]
        for cp in ocps:
            cp.wait()

    return gather_kernel(X3, emb_E, emb_R)


_CONTRACT_1_1 = (((1,), (1,)), ((), ()))


def _mlp_body(ehs, ets, els, xlit, wlitT, blit, w1T, b1, w2r, b2, out):
    elit = jnp.dot(xlit[...], wlitT[...],
                   preferred_element_type=jnp.float32) + blit[...]
    phi = jnp.concatenate([ehs[...], ets[...], els[...], elit], axis=1)
    h = jnp.maximum(
        jnp.dot(phi.astype(jnp.bfloat16), w1T[...],
                preferred_element_type=jnp.float32) + b1[...],
        0.0)
    y = jnp.sum(h * w2r[...], axis=1, keepdims=True) + b2[...]
    out[...] = jax.nn.sigmoid(y)


def _mlp_call(e_hs, e_ts, e_ls, xlit, wlitT, blit, w1T, b1, w2r, b2):
    M = e_hs.shape[0]
    na = xlit.shape[1]
    bm = 4096
    grid = (M // bm,)
    return pl.pallas_call(
        _mlp_body,
        grid=grid,
        in_specs=[
            pl.BlockSpec((bm, 128), lambda i: (i, 0)),
            pl.BlockSpec((bm, 128), lambda i: (i, 0)),
            pl.BlockSpec((bm, 128), lambda i: (i, 0)),
            pl.BlockSpec((bm, na), lambda i: (i, 0)),
            pl.BlockSpec((na, 128), lambda i: (0, 0)),
            pl.BlockSpec((1, 128), lambda i: (0, 0)),
            pl.BlockSpec((512, 1024), lambda i: (0, 0)),
            pl.BlockSpec((1, 1024), lambda i: (0, 0)),
            pl.BlockSpec((1, 1024), lambda i: (0, 0)),
            pl.BlockSpec((1, 1), lambda i: (0, 0)),
        ],
        out_specs=pl.BlockSpec((bm, 1), lambda i: (i, 0)),
        out_shape=jax.ShapeDtypeStruct((M, 1), jnp.float32),
    )(e_hs, e_ts, e_ls, xlit, wlitT, blit, w1T, b1, w2r, b2)


_NSPLIT = 4


def kernel(X, X_lit, emb_E, emb_R, W_lit, b_lit, W1, b1, W2, b2):
    M = X.shape[1]
    Mh = M // _NSPLIT
    Xi = X.astype(jnp.int32)
    xlit_b = X_lit.astype(jnp.bfloat16)
    wlitT_b = W_lit.T.astype(jnp.bfloat16)
    blit = b_lit.reshape(1, -1)
    w1T_b = W1.T.astype(jnp.bfloat16)
    b1r = b1.reshape(1, -1)
    b2r = b2.reshape(1, 1)
    outs = []
    for s in range(_NSPLIT):
        Xs = Xi[:, s * Mh:(s + 1) * Mh].reshape(3, Mh // 128, 128)
        e_hs, e_ls, e_ts = _sc_gather(Xs, emb_E, emb_R)
        outs.append(_mlp_call(
            e_hs, e_ts, e_ls, xlit_b[s * Mh:(s + 1) * Mh], wlitT_b,
            blit, w1T_b, b1r, W2, b2r))
    return jnp.concatenate(outs, axis=0)


# speedup vs baseline: 1.1960x; 1.1960x over previous
import functools

import jax
import jax.numpy as jnp
from jax import lax
from jax.experimental import pallas as pl
from jax.experimental.pallas import tpu as pltpu
from jax.experimental.pallas import tpu_sc as plsc

_NC = 2
_NS = 16
_NW = _NC * _NS


def _sc_gather(X3, emb_E, emb_R):
    M = X3.shape[1] * X3.shape[2]
    D = emb_E.shape[1]
    bpw = M // _NW
    nchunk = bpw // 128

    mesh = plsc.VectorSubcoreMesh(core_axis_name="c", subcore_axis_name="s")

    @functools.partial(
        pl.kernel,
        mesh=mesh,
        out_type=[jax.ShapeDtypeStruct((M, D), jnp.float32) for _ in range(3)],
        scratch_types=[
            pltpu.VMEM((3, nchunk, 128), jnp.int32),
            pltpu.VMEM((3 * bpw, D), jnp.float32),
            pltpu.SemaphoreType.DMA,
            pltpu.SemaphoreType.DMA,
        ],
    )
    def gather_kernel(x_hbm, e_hbm, r_hbm, out_hs, out_ls, out_ts,
                      idx_v, rows_v, gsem, osem):
        wid = lax.axis_index("s") * _NC + lax.axis_index("c")
        rowbase = wid * nchunk
        base = wid * bpw
        pltpu.sync_copy(x_hbm.at[:, pl.ds(rowbase, nchunk), :], idx_v)
        tabs = (e_hbm, r_hbm, e_hbm)
        cps = []
        for seg in range(3):
            for j in range(nchunk):
                cps.append(pltpu.async_copy(
                    tabs[seg].at[idx_v.at[seg, j]],
                    rows_v.at[pl.ds((seg * nchunk + j) * 128, 128)], gsem))
        for cp in cps:
            cp.wait()
        outs = (out_hs, out_ls, out_ts)
        ocps = [
            pltpu.async_copy(rows_v.at[pl.ds(seg * bpw, bpw)],
                             outs[seg].at[pl.ds(base, bpw)], osem)
            for seg in range(3)
        ]
        for cp in ocps:
            cp.wait()

    return gather_kernel(X3, emb_E, emb_R)


_CONTRACT_1_1 = (((1,), (1,)), ((), ()))


def _mlp_body(ehs, ets, els, xlit, wlitT, blit, w1T, b1, w2r, b2, out):
    elit = jnp.dot(xlit[...], wlitT[...],
                   preferred_element_type=jnp.float32) + blit[...]
    phi = jnp.concatenate([ehs[...], ets[...], els[...], elit], axis=1)
    h = jnp.maximum(
        jnp.dot(phi.astype(jnp.bfloat16), w1T[...],
                preferred_element_type=jnp.float32) + b1[...],
        0.0)
    y = jnp.sum(h * w2r[...], axis=1, keepdims=True) + b2[...]
    out[...] = jax.nn.sigmoid(y)


def _mlp_call(e_hs, e_ts, e_ls, xlit, wlitT, blit, w1T, b1, w2r, b2):
    M = e_hs.shape[0]
    na = xlit.shape[1]
    bm = 4096
    grid = (M // bm,)
    return pl.pallas_call(
        _mlp_body,
        grid=grid,
        in_specs=[
            pl.BlockSpec((bm, 128), lambda i: (i, 0)),
            pl.BlockSpec((bm, 128), lambda i: (i, 0)),
            pl.BlockSpec((bm, 128), lambda i: (i, 0)),
            pl.BlockSpec((bm, na), lambda i: (i, 0)),
            pl.BlockSpec((na, 128), lambda i: (0, 0)),
            pl.BlockSpec((1, 128), lambda i: (0, 0)),
            pl.BlockSpec((512, 1024), lambda i: (0, 0)),
            pl.BlockSpec((1, 1024), lambda i: (0, 0)),
            pl.BlockSpec((1, 1024), lambda i: (0, 0)),
            pl.BlockSpec((1, 1), lambda i: (0, 0)),
        ],
        out_specs=pl.BlockSpec((bm, 1), lambda i: (i, 0)),
        out_shape=jax.ShapeDtypeStruct((M, 1), jnp.float32),
    )(e_hs, e_ts, e_ls, xlit, wlitT, blit, w1T, b1, w2r, b2)


_NSPLIT = 2


def kernel(X, X_lit, emb_E, emb_R, W_lit, b_lit, W1, b1, W2, b2):
    M = X.shape[1]
    Mh = M // _NSPLIT
    Xi = X.astype(jnp.int32)
    xlit_b = X_lit.astype(jnp.bfloat16)
    wlitT_b = W_lit.T.astype(jnp.bfloat16)
    blit = b_lit.reshape(1, -1)
    w1T_b = W1.T.astype(jnp.bfloat16)
    b1r = b1.reshape(1, -1)
    b2r = b2.reshape(1, 1)
    outs = []
    for s in range(_NSPLIT):
        Xs = Xi[:, s * Mh:(s + 1) * Mh].reshape(3, Mh // 128, 128)
        e_hs, e_ls, e_ts = _sc_gather(Xs, emb_E, emb_R)
        outs.append(_mlp_call(
            e_hs, e_ts, e_ls, xlit_b[s * Mh:(s + 1) * Mh], wlitT_b,
            blit, w1T_b, b1r, W2, b2r))
    return jnp.concatenate(outs, axis=0)


# speedup vs baseline: 1.2321x; 1.0302x over previous
import functools

import jax
import jax.numpy as jnp
from jax import lax
from jax.experimental import pallas as pl
from jax.experimental.pallas import tpu as pltpu
from jax.experimental.pallas import tpu_sc as plsc

_NC = 2
_NS = 16
_NW = _NC * _NS


def _sc_gather(X3, emb_E, emb_R):
    M = X3.shape[1] * X3.shape[2]
    D = emb_E.shape[1]
    bpw = M // _NW
    nchunk = bpw // 128

    mesh = plsc.VectorSubcoreMesh(core_axis_name="c", subcore_axis_name="s")

    @functools.partial(
        pl.kernel,
        mesh=mesh,
        out_type=[jax.ShapeDtypeStruct((M, D), jnp.float32) for _ in range(3)],
        scratch_types=[
            pltpu.VMEM((3, nchunk, 128), jnp.int32),
            pltpu.VMEM((3 * bpw, D), jnp.float32),
            pltpu.SemaphoreType.DMA,
            pltpu.SemaphoreType.DMA,
        ],
    )
    def gather_kernel(x_hbm, e_hbm, r_hbm, out_hs, out_ls, out_ts,
                      idx_v, rows_v, gsem, osem):
        wid = lax.axis_index("s") * _NC + lax.axis_index("c")
        rowbase = wid * nchunk
        base = wid * bpw
        pltpu.sync_copy(x_hbm.at[:, pl.ds(rowbase, nchunk), :], idx_v)
        tabs = (e_hbm, r_hbm, e_hbm)
        cps = []
        for seg in range(3):
            for j in range(nchunk):
                cps.append(pltpu.async_copy(
                    tabs[seg].at[idx_v.at[seg, j]],
                    rows_v.at[pl.ds((seg * nchunk + j) * 128, 128)], gsem))
        for cp in cps:
            cp.wait()
        outs = (out_hs, out_ls, out_ts)
        ocps = [
            pltpu.async_copy(rows_v.at[pl.ds(seg * bpw, bpw)],
                             outs[seg].at[pl.ds(base, bpw)], osem)
            for seg in range(3)
        ]
        for cp in ocps:
            cp.wait()

    return gather_kernel(X3, emb_E, emb_R)


_CONTRACT_1_1 = (((1,), (1,)), ((), ()))


def _mlp_body(ehs, ets, els, xlit, wlitT, blit, w1T, b1, w2r, b2, out):
    elit = jnp.dot(xlit[...], wlitT[...],
                   preferred_element_type=jnp.float32) + blit[...]
    phi = jnp.concatenate([ehs[...], ets[...], els[...], elit], axis=1)
    h = jnp.maximum(
        jnp.dot(phi.astype(jnp.bfloat16), w1T[...],
                preferred_element_type=jnp.float32) + b1[...],
        0.0)
    y = jnp.sum(h * w2r[...], axis=1, keepdims=True) + b2[...]
    out[...] = jax.nn.sigmoid(y)


def _mlp_call(e_hs, e_ts, e_ls, xlit, wlitT, blit, w1T, b1, w2r, b2):
    M = e_hs.shape[0]
    na = xlit.shape[1]
    bm = 2048
    grid = (M // bm,)
    return pl.pallas_call(
        _mlp_body,
        grid=grid,
        in_specs=[
            pl.BlockSpec((bm, 128), lambda i: (i, 0)),
            pl.BlockSpec((bm, 128), lambda i: (i, 0)),
            pl.BlockSpec((bm, 128), lambda i: (i, 0)),
            pl.BlockSpec((bm, na), lambda i: (i, 0)),
            pl.BlockSpec((na, 128), lambda i: (0, 0)),
            pl.BlockSpec((1, 128), lambda i: (0, 0)),
            pl.BlockSpec((512, 1024), lambda i: (0, 0)),
            pl.BlockSpec((1, 1024), lambda i: (0, 0)),
            pl.BlockSpec((1, 1024), lambda i: (0, 0)),
            pl.BlockSpec((1, 1), lambda i: (0, 0)),
        ],
        out_specs=pl.BlockSpec((bm, 1), lambda i: (i, 0)),
        out_shape=jax.ShapeDtypeStruct((M, 1), jnp.float32),
    )(e_hs, e_ts, e_ls, xlit, wlitT, blit, w1T, b1, w2r, b2)


_NSPLIT = 2


def kernel(X, X_lit, emb_E, emb_R, W_lit, b_lit, W1, b1, W2, b2):
    M = X.shape[1]
    Mh = M // _NSPLIT
    Xi = X.astype(jnp.int32)
    xlit_b = X_lit.astype(jnp.bfloat16)
    wlitT_b = W_lit.T.astype(jnp.bfloat16)
    blit = b_lit.reshape(1, -1)
    w1T_b = W1.T.astype(jnp.bfloat16)
    b1r = b1.reshape(1, -1)
    b2r = b2.reshape(1, 1)
    outs = []
    for s in range(_NSPLIT):
        Xs = Xi[:, s * Mh:(s + 1) * Mh].reshape(3, Mh // 128, 128)
        e_hs, e_ls, e_ts = _sc_gather(Xs, emb_E, emb_R)
        outs.append(_mlp_call(
            e_hs, e_ts, e_ls, xlit_b[s * Mh:(s + 1) * Mh], wlitT_b,
            blit, w1T_b, b1r, W2, b2r))
    return jnp.concatenate(outs, axis=0)
